# baseline (speedup 1.0000x reference)
import functools

import jax
import jax.numpy as jnp
from jax import lax
from jax.experimental import pallas as pl
from jax.experimental.pallas import tpu as pltpu
from jax.experimental.pallas import tpu_sc as plsc

T = 512
B = 16
C = 128
W = 16
NEG_INF = -1e30
BIG = 1 << 30
PK = 4096


def _logaddexp(a, b):
    m = jnp.maximum(a, b)
    y = jnp.exp(-jnp.abs(a - b))
    s = y / (y + 2.0)
    q = s * s
    p = jnp.float32(1.0 / 13)
    for coef in (1.0 / 11, 1.0 / 9, 1.0 / 7, 1.0 / 5, 1.0 / 3, 1.0):
        p = p * q + jnp.float32(coef)
    return m + (s + s) * p


def _scalar(x):
    if getattr(x, "ndim", 0) == 0:
        return x
    return x[0]


def _sc_body(data_hbm, len_hbm, probs_hbm, lens_hbm, labels_hbm,
             data_v, bp_v, cbuf_v, ibuf_v, stb_v, stn_v, lens_v, last_v,
             selv_v, seli_v, lenall_v, clist_v, slot_v, lab_v, outf_v,
             outi_v):
    wid = lax.axis_index("s") * 2 + lax.axis_index("c")

    @pl.when(wid < B)
    def _():
        b = wid
        pltpu.sync_copy(data_hbm.at[b], data_v)
        pltpu.sync_copy(len_hbm, lenall_v)
        iota = lax.iota(jnp.int32, 16)
        neg_inf = jnp.full((16,), NEG_INF, jnp.float32)
        bvec = jnp.full((16,), b, jnp.int32)
        L = plsc.load_gather(lenall_v, [bvec])[0]

        def step(t, carry):
            lpb, lpnb, tot, lens, last, slot = carry
            tvec = jnp.full((16,), t, jnp.int32)
            lp0 = plsc.load_gather(data_v, [tvec, jnp.zeros((16,), jnp.int32)])
            lp_last = plsc.load_gather(data_v, [tvec, last])
            stay_lpb = tot + lp0
            stay_lpnb = lpnb + lp_last
            stay = _logaddexp(stay_lpb, stay_lpnb)
            stay_packed = slot * PK + iota
            val, idx = plsc.sort_key_val(stay, stay_packed)

            stb_v[...] = stay_lpb
            stn_v[...] = stay_lpnb
            lens_v[...] = lens
            last_v[...] = last
            slot_v[...] = slot

            pbase = (16 + slot * 128) * PK + (16 + iota * 128)

            bmax = jnp.max(tot)
            wstar = plsc.all_reduce_ffs(tot == bmax)
            last_star = plsc.load_gather(last_v, [wstar])
            slot_star = plsc.load_gather(slot_v, [wstar])
            star_pbase = (16 + slot_star * 128) * PK + 16 + wstar * 128

            def skippable_merge(v, i, chv, chp):
                def do(_):
                    bv, bi = plsc.sort_key_val(chv, chp, descending=True)
                    take = bv > v
                    nv, ni = plsc.sort_key_val(jnp.where(take, bv, v),
                                               jnp.where(take, bi, i))
                    return (nv, ni)

                return lax.cond(jnp.max(chv) > v[0], do, lambda _: (v, i), 0)

            def premerge_body(g, vi):
                v, i = vi
                ci = g * 16 + iota
                lp16 = plsc.load_gather(data_v, [tvec, ci])
                chv = jnp.where((ci != 0) & (ci != last_star),
                                bmax + lp16, neg_inf)
                chp = star_pbase + ci * (PK + 1)
                return skippable_merge(v, i, chv, chp)

            val, idx = lax.fori_loop(0, C // 16, premerge_body, (val, idx),
                                     unroll=8)
            tau = val[0]

            def pre_body(g, ns):
                cvec = g * 16 + iota
                lp16 = plsc.load_gather(data_v, [tvec, cvec])
                m = (lp16 + bmax >= tau) & (cvec > 0)
                cnt = _scalar(plsc.all_reduce_population_count(m))
                plsc.store_compressed(clist_v.at[pl.ds(ns, 16)], cvec, mask=m)
                return ns + cnt

            ns = lax.fori_loop(0, C // 16, pre_body, jnp.int32(0), unroll=8)

            def class_body(j, off):
                cvec = plsc.load_gather(clist_v, [jnp.full((16,), j, jnp.int32)])
                lp_c = plsc.load_gather(data_v, [tvec, cvec])
                cand = jnp.where(last == cvec, lpb, tot) + lp_c
                msk = (cand >= tau) & ((iota != wstar) | (cvec == last_star))
                cnt = _scalar(plsc.all_reduce_population_count(msk))
                plsc.store_compressed(cbuf_v.at[pl.ds(off, 16)], cand, mask=msk)
                plsc.store_compressed(ibuf_v.at[pl.ds(off, 16)],
                                      pbase + cvec * (PK + 1), mask=msk)
                return off + cnt

            off = lax.fori_loop(0, ns, class_body, jnp.int32(0))
            nch = (off + 15) >> 4

            def merge_body(j, vi):
                v, i = vi
                base = j * 16
                bv = cbuf_v[pl.ds(base, 16)]
                bi = ibuf_v[pl.ds(base, 16)]
                bv = jnp.where(base + iota < off, bv, neg_inf)
                return skippable_merge(v, i, bv, bi)

            val, idx = lax.fori_loop(0, nch, merge_body, (val, idx))

            v16 = val[0]
            n_sel_eq = _scalar(plsc.all_reduce_population_count(val == v16))

            def eq_body(j, n):
                bv = cbuf_v[pl.ds(j * 16, 16)]
                m = (bv == v16) & (j * 16 + iota < off)
                return n + _scalar(plsc.all_reduce_population_count(m))

            def eq_star_body(g, n):
                ci = g * 16 + iota
                lp16 = plsc.load_gather(data_v, [tvec, ci])
                m = (bmax + lp16 == v16) & (ci != 0) & (ci != last_star)
                return n + _scalar(plsc.all_reduce_population_count(m))

            n_all_eq = lax.fori_loop(
                0, nch, eq_body,
                lax.fori_loop(
                    0, C // 16, eq_star_body,
                    _scalar(plsc.all_reduce_population_count(stay == v16)),
                    unroll=8))

            def fixup(cur_idx):
                big = jnp.full((16,), BIG, jnp.int32)
                ch = jnp.where(stay == v16, stay_packed, big)
                ch, _ = plsc.sort_key_val(ch, ch, descending=True)
                fixv, _ = plsc.sort_key_val(jnp.minimum(big, ch), ch)

                def fix_body(j, fv):
                    bv = cbuf_v[pl.ds(j * 16, 16)]
                    bi = ibuf_v[pl.ds(j * 16, 16)]
                    m = (bv == v16) & (j * 16 + iota < off)
                    ch2 = jnp.where(m, bi, big)
                    ch2, _ = plsc.sort_key_val(ch2, ch2, descending=True)
                    nf = jnp.minimum(fv, ch2)
                    nf, _ = plsc.sort_key_val(nf, nf)
                    return nf

                def fix_star_body(g, fv):
                    ci = g * 16 + iota
                    lp16 = plsc.load_gather(data_v, [tvec, ci])
                    m = (bmax + lp16 == v16) & (ci != 0) & (ci != last_star)
                    ch2 = jnp.where(m, star_pbase + ci * (PK + 1), big)
                    ch2, _ = plsc.sort_key_val(ch2, ch2, descending=True)
                    nf = jnp.minimum(fv, ch2)
                    nf, _ = plsc.sort_key_val(nf, nf)
                    return nf

                fixv = lax.fori_loop(0, nch, fix_body,
                                     lax.fori_loop(0, C // 16, fix_star_body,
                                                   fixv))
                return jnp.where(iota < n_sel_eq, fixv, cur_idx)

            idx = lax.cond(n_all_eq != n_sel_eq, fixup, lambda i: i, idx)

            selv_v[...] = val
            seli_v[...] = idx

            def rank_slow():
                def rank_body(k, r):
                    kvec = jnp.full((16,), k, jnp.int32)
                    vk = plsc.load_gather(selv_v, [kvec])
                    pk = plsc.load_gather(seli_v, [kvec])
                    beats = (vk > val) | ((vk == val) & (pk < idx))
                    return r + beats.astype(jnp.int32)

                return lax.fori_loop(0, 16, rank_body,
                                     jnp.zeros((16,), jnp.int32), unroll=4)

            prev = plsc.load_gather(selv_v, [jnp.maximum(iota - 1, 0)])
            ndup = _scalar(plsc.all_reduce_population_count(
                (prev == val) & (iota > 0)))
            rank = lax.cond(ndup == 0, lambda: 15 - iota, rank_slow)

            midx = idx & (PK - 1)
            is_stay = midx < 16
            parent = jnp.where(is_stay, midx, (midx - 16) >> 7)
            c_new = jnp.where(is_stay, 0, (midx - 16) & 127)
            g_slpb = plsc.load_gather(stb_v, [parent])
            g_slpnb = plsc.load_gather(stn_v, [parent])
            g_lens = plsc.load_gather(lens_v, [parent])
            g_last = plsc.load_gather(last_v, [parent])
            n_lpb = jnp.where(is_stay, g_slpb, neg_inf)
            n_lpnb = jnp.where(is_stay, g_slpnb, val)
            n_lens = jnp.where(is_stay, g_lens, g_lens + 1)
            n_last = jnp.where(is_stay, g_last, c_new)
            row = bp_v[t >> 1, :]
            odd = (t & 1) == 1
            bp_v[t >> 1, :] = jnp.where(odd, row | (midx << 16), midx)
            return (n_lpb, n_lpnb, val, n_lens, n_last, rank)

        lpb0 = jnp.where(iota == 0, 0.0, neg_inf)
        lpb, lpnb, total, lens, last, slot = lax.fori_loop(
            0, L, step, (lpb0, neg_inf, lpb0, jnp.zeros((16,), jnp.int32),
                         jnp.zeros((16,), jnp.int32), iota))

        m = jnp.max(total)
        key = jnp.where(total == m, slot, jnp.full((16,), BIG, jnp.int32))
        best = _scalar(plsc.all_reduce_ffs(key == jnp.min(key)))
        lens_v[...] = lens
        blen = plsc.load_gather(lens_v, [jnp.full((16,), best, jnp.int32)])[0]
        outf_v[...] = jnp.full((16,), 0.0, jnp.float32) - m
        outi_v[...] = jnp.full((16,), blen, jnp.int32)

        def zero_body(i, _):
            lab_v[pl.ds(i * 16, 16)] = jnp.zeros((16,), jnp.int32)
            return 0

        lax.fori_loop(0, 33, zero_body, 0)

        lane0 = iota == 0

        def bt_body(i, carry):
            cur, pos = carry
            tt = L - 1 - i
            iv32 = plsc.load_gather(
                bp_v, [jnp.full((16,), tt >> 1, jnp.int32),
                       jnp.full((16,), cur, jnp.int32)])[0]
            iv = jnp.where((tt & 1) == 1, iv32 >> 16, iv32 & 0xFFFF)
            stayf = iv < 16
            c = jnp.where(stayf, 0, (iv - 16) & 127)
            ncur = jnp.where(stayf, iv, (iv - 16) >> 7)
            ext = c != 0
            npos = pos - ext.astype(jnp.int32)
            addr = jnp.where(ext, npos, 520)
            plsc.store_scatter(lab_v, [jnp.full((16,), addr, jnp.int32)],
                               jnp.full((16,), c, jnp.int32), mask=lane0)
            return (ncur, npos)

        lax.fori_loop(0, L, bt_body, (best, blen))

        pltpu.sync_copy(lab_v.at[pl.ds(0, T)], labels_hbm.at[b])
        pltpu.sync_copy(outf_v, probs_hbm.at[b])
        pltpu.sync_copy(outi_v, lens_hbm.at[b])


@jax.jit
def _sc_decode(data_bt, data_length):
    mesh = plsc.VectorSubcoreMesh(core_axis_name="c", subcore_axis_name="s")
    fn = functools.partial(
        pl.kernel,
        out_type=[
            jax.ShapeDtypeStruct((B, 16), jnp.float32),
            jax.ShapeDtypeStruct((B, 16), jnp.int32),
            jax.ShapeDtypeStruct((B, T), jnp.int32),
        ],
        mesh=mesh,
        compiler_params=pltpu.CompilerParams(needs_layout_passes=False),
        scratch_types=[
            pltpu.VMEM((T, C), jnp.float32),
            pltpu.VMEM((T // 2, W), jnp.int32),
            pltpu.VMEM((2080,), jnp.float32),
            pltpu.VMEM((2080,), jnp.int32),
            pltpu.VMEM((16,), jnp.float32),
            pltpu.VMEM((16,), jnp.float32),
            pltpu.VMEM((16,), jnp.int32),
            pltpu.VMEM((16,), jnp.int32),
            pltpu.VMEM((16,), jnp.float32),
            pltpu.VMEM((16,), jnp.int32),
            pltpu.VMEM((16,), jnp.int32),
            pltpu.VMEM((144,), jnp.int32),
            pltpu.VMEM((16,), jnp.int32),
            pltpu.VMEM((528,), jnp.int32),
            pltpu.VMEM((16,), jnp.float32),
            pltpu.VMEM((16,), jnp.int32),
        ],
    )(_sc_body)
    return fn(data_bt, data_length)


def kernel(data, data_length):
    data_bt = jnp.transpose(data, (1, 0, 2))
    probs8, lens8, labels = _sc_decode(data_bt, data_length.astype(jnp.int32))
    return probs8[:, :1], lens8[:, :1], labels.reshape(B, 1, T)


# speedup vs baseline: 1.5360x; 1.5360x over previous
import functools

import jax
import jax.numpy as jnp
from jax import lax
from jax.experimental import pallas as pl
from jax.experimental.pallas import tpu as pltpu
from jax.experimental.pallas import tpu_sc as plsc

T = 512
B = 16
C = 128
W = 16
NEG_INF = -1e30
BIG = 1 << 30
PK = 4096


def _logaddexp(a, b):
    m = jnp.maximum(a, b)
    y = jnp.exp(-jnp.abs(a - b))
    s = y / (y + 2.0)
    q = s * s
    p = jnp.float32(1.0 / 13)
    for coef in (1.0 / 11, 1.0 / 9, 1.0 / 7, 1.0 / 5, 1.0 / 3, 1.0):
        p = p * q + jnp.float32(coef)
    return m + (s + s) * p


def _scalar(x):
    if getattr(x, "ndim", 0) == 0:
        return x
    return x[0]


def _sc_body(data_hbm, len_hbm, probs_hbm, lens_hbm, labels_hbm,
             data_v, bp_v, cbuf_v, ibuf_v, stb_v, stn_v, lens_v, last_v,
             selv_v, seli_v, lenall_v, clist_v, slot_v, lab_v, outf_v,
             outi_v):
    wid = lax.axis_index("s") * 2 + lax.axis_index("c")

    @pl.when(wid < B)
    def _():
        b = wid
        pltpu.sync_copy(data_hbm.at[b], data_v)
        pltpu.sync_copy(len_hbm, lenall_v)
        iota = lax.iota(jnp.int32, 16)
        neg_inf = jnp.full((16,), NEG_INF, jnp.float32)
        bvec = jnp.full((16,), b, jnp.int32)
        L = plsc.load_gather(lenall_v, [bvec])[0]

        def step(t, carry):
            lpb, lpnb, tot, lens, last, slot = carry
            tvec = jnp.full((16,), t, jnp.int32)
            lp0 = plsc.load_gather(data_v, [tvec, jnp.zeros((16,), jnp.int32)])
            lp_last = plsc.load_gather(data_v, [tvec, last])
            stay_lpb = tot + lp0
            stay_lpnb = lpnb + lp_last
            stay = _logaddexp(stay_lpb, stay_lpnb)
            stay_packed = slot * PK + iota
            val, idx = plsc.sort_key_val(stay, stay_packed)

            stb_v[...] = stay_lpb
            stn_v[...] = stay_lpnb
            lens_v[...] = lens
            last_v[...] = last
            slot_v[...] = slot

            pbase = (16 + slot * 128) * PK + (16 + iota * 128)

            bmax = jnp.max(tot)
            wstar = plsc.all_reduce_ffs(tot == bmax)
            last_star = plsc.load_gather(last_v, [wstar])
            slot_star = plsc.load_gather(slot_v, [wstar])
            star_pbase = (16 + slot_star * 128) * PK + 16 + wstar * 128

            def premerge_body(g, vi):
                v, i = vi
                ci = g * 16 + iota
                lp16 = plsc.load_gather(data_v, [tvec, ci])
                chv = jnp.where((ci != 0) & (ci != last_star),
                                bmax + lp16, neg_inf)
                chp = star_pbase + ci * (PK + 1)
                bv, bi = plsc.sort_key_val(chv, chp, descending=True)
                take = bv > v
                nv, ni = plsc.sort_key_val(jnp.where(take, bv, v),
                                           jnp.where(take, bi, i))
                return (nv, ni)

            val, idx = lax.fori_loop(0, C // 16, premerge_body, (val, idx),
                                     unroll=8)
            tau = val[0]

            def pre_body(g, ns):
                cvec = g * 16 + iota
                lp16 = plsc.load_gather(data_v, [tvec, cvec])
                m = (lp16 + bmax >= tau) & (cvec > 0)
                cnt = _scalar(plsc.all_reduce_population_count(m))
                plsc.store_compressed(clist_v.at[pl.ds(ns, 16)], cvec, mask=m)
                return ns + cnt

            ns = lax.fori_loop(0, C // 16, pre_body, jnp.int32(0), unroll=8)

            def class_body(j, _):
                cvec = plsc.load_gather(clist_v, [jnp.full((16,), j, jnp.int32)])
                lp_c = plsc.load_gather(data_v, [tvec, cvec])
                cand = jnp.where(last == cvec, lpb, tot) + lp_c
                msk = (cand >= tau) & ((iota != wstar) | (cvec == last_star))
                cbuf_v[pl.ds(j * 16, 16)] = jnp.where(msk, cand, neg_inf)
                ibuf_v[pl.ds(j * 16, 16)] = pbase + cvec * (PK + 1)
                return 0

            lax.fori_loop(0, ns, class_body, 0)
            nch = ns

            def merge_body(j, vi):
                v, i = vi
                base = j * 16
                bv = cbuf_v[pl.ds(base, 16)]
                bi = ibuf_v[pl.ds(base, 16)]
                bv, bi = plsc.sort_key_val(bv, bi, descending=True)
                take = bv > v
                nv, ni = plsc.sort_key_val(jnp.where(take, bv, v),
                                           jnp.where(take, bi, i))
                return (nv, ni)

            val, idx = lax.fori_loop(0, nch, merge_body, (val, idx))

            v16 = val[0]
            n_sel_eq = _scalar(plsc.all_reduce_population_count(val == v16))

            def eq_body(j, n):
                bv = cbuf_v[pl.ds(j * 16, 16)]
                m = bv == v16
                return n + _scalar(plsc.all_reduce_population_count(m))

            def eq_star_body(g, n):
                ci = g * 16 + iota
                lp16 = plsc.load_gather(data_v, [tvec, ci])
                m = (bmax + lp16 == v16) & (ci != 0) & (ci != last_star)
                return n + _scalar(plsc.all_reduce_population_count(m))

            n_all_eq = lax.fori_loop(
                0, nch, eq_body,
                lax.fori_loop(
                    0, C // 16, eq_star_body,
                    _scalar(plsc.all_reduce_population_count(stay == v16)),
                    unroll=8))

            def fixup(cur_idx):
                big = jnp.full((16,), BIG, jnp.int32)
                ch = jnp.where(stay == v16, stay_packed, big)
                ch, _ = plsc.sort_key_val(ch, ch, descending=True)
                fixv, _ = plsc.sort_key_val(jnp.minimum(big, ch), ch)

                def fix_body(j, fv):
                    bv = cbuf_v[pl.ds(j * 16, 16)]
                    bi = ibuf_v[pl.ds(j * 16, 16)]
                    m = bv == v16
                    ch2 = jnp.where(m, bi, big)
                    ch2, _ = plsc.sort_key_val(ch2, ch2, descending=True)
                    nf = jnp.minimum(fv, ch2)
                    nf, _ = plsc.sort_key_val(nf, nf)
                    return nf

                def fix_star_body(g, fv):
                    ci = g * 16 + iota
                    lp16 = plsc.load_gather(data_v, [tvec, ci])
                    m = (bmax + lp16 == v16) & (ci != 0) & (ci != last_star)
                    ch2 = jnp.where(m, star_pbase + ci * (PK + 1), big)
                    ch2, _ = plsc.sort_key_val(ch2, ch2, descending=True)
                    nf = jnp.minimum(fv, ch2)
                    nf, _ = plsc.sort_key_val(nf, nf)
                    return nf

                fixv = lax.fori_loop(0, nch, fix_body,
                                     lax.fori_loop(0, C // 16, fix_star_body,
                                                   fixv))
                return jnp.where(iota < n_sel_eq, fixv, cur_idx)

            idx = lax.cond(n_all_eq != n_sel_eq, fixup, lambda i: i, idx)

            selv_v[...] = val
            seli_v[...] = idx

            def rank_slow():
                def rank_body(k, r):
                    kvec = jnp.full((16,), k, jnp.int32)
                    vk = plsc.load_gather(selv_v, [kvec])
                    pk = plsc.load_gather(seli_v, [kvec])
                    beats = (vk > val) | ((vk == val) & (pk < idx))
                    return r + beats.astype(jnp.int32)

                return lax.fori_loop(0, 16, rank_body,
                                     jnp.zeros((16,), jnp.int32), unroll=4)

            prev = plsc.load_gather(selv_v, [jnp.maximum(iota - 1, 0)])
            ndup = _scalar(plsc.all_reduce_population_count(
                (prev == val) & (iota > 0)))
            rank = lax.cond(ndup == 0, lambda: 15 - iota, rank_slow)

            midx = idx & (PK - 1)
            is_stay = midx < 16
            parent = jnp.where(is_stay, midx, (midx - 16) >> 7)
            c_new = jnp.where(is_stay, 0, (midx - 16) & 127)
            g_slpb = plsc.load_gather(stb_v, [parent])
            g_slpnb = plsc.load_gather(stn_v, [parent])
            g_lens = plsc.load_gather(lens_v, [parent])
            g_last = plsc.load_gather(last_v, [parent])
            n_lpb = jnp.where(is_stay, g_slpb, neg_inf)
            n_lpnb = jnp.where(is_stay, g_slpnb, val)
            n_lens = jnp.where(is_stay, g_lens, g_lens + 1)
            n_last = jnp.where(is_stay, g_last, c_new)
            row = bp_v[t >> 1, :]
            odd = (t & 1) == 1
            bp_v[t >> 1, :] = jnp.where(odd, row | (midx << 16), midx)
            return (n_lpb, n_lpnb, val, n_lens, n_last, rank)

        lpb0 = jnp.where(iota == 0, 0.0, neg_inf)
        lpb, lpnb, total, lens, last, slot = lax.fori_loop(
            0, L, step, (lpb0, neg_inf, lpb0, jnp.zeros((16,), jnp.int32),
                         jnp.zeros((16,), jnp.int32), iota))

        m = jnp.max(total)
        key = jnp.where(total == m, slot, jnp.full((16,), BIG, jnp.int32))
        best = _scalar(plsc.all_reduce_ffs(key == jnp.min(key)))
        lens_v[...] = lens
        blen = plsc.load_gather(lens_v, [jnp.full((16,), best, jnp.int32)])[0]
        outf_v[...] = jnp.full((16,), 0.0, jnp.float32) - m
        outi_v[...] = jnp.full((16,), blen, jnp.int32)

        def zero_body(i, _):
            lab_v[pl.ds(i * 16, 16)] = jnp.zeros((16,), jnp.int32)
            return 0

        lax.fori_loop(0, 33, zero_body, 0)

        lane0 = iota == 0

        def bt_body(i, carry):
            cur, pos = carry
            tt = L - 1 - i
            iv32 = plsc.load_gather(
                bp_v, [jnp.full((16,), tt >> 1, jnp.int32),
                       jnp.full((16,), cur, jnp.int32)])[0]
            iv = jnp.where((tt & 1) == 1, iv32 >> 16, iv32 & 0xFFFF)
            stayf = iv < 16
            c = jnp.where(stayf, 0, (iv - 16) & 127)
            ncur = jnp.where(stayf, iv, (iv - 16) >> 7)
            ext = c != 0
            npos = pos - ext.astype(jnp.int32)
            addr = jnp.where(ext, npos, 520)
            plsc.store_scatter(lab_v, [jnp.full((16,), addr, jnp.int32)],
                               jnp.full((16,), c, jnp.int32), mask=lane0)
            return (ncur, npos)

        lax.fori_loop(0, L, bt_body, (best, blen))

        pltpu.sync_copy(lab_v.at[pl.ds(0, T)], labels_hbm.at[b])
        pltpu.sync_copy(outf_v, probs_hbm.at[b])
        pltpu.sync_copy(outi_v, lens_hbm.at[b])


@jax.jit
def _sc_decode(data_bt, data_length):
    mesh = plsc.VectorSubcoreMesh(core_axis_name="c", subcore_axis_name="s")
    fn = functools.partial(
        pl.kernel,
        out_type=[
            jax.ShapeDtypeStruct((B, 16), jnp.float32),
            jax.ShapeDtypeStruct((B, 16), jnp.int32),
            jax.ShapeDtypeStruct((B, T), jnp.int32),
        ],
        mesh=mesh,
        compiler_params=pltpu.CompilerParams(needs_layout_passes=False),
        scratch_types=[
            pltpu.VMEM((T, C), jnp.float32),
            pltpu.VMEM((T // 2, W), jnp.int32),
            pltpu.VMEM((2080,), jnp.float32),
            pltpu.VMEM((2080,), jnp.int32),
            pltpu.VMEM((16,), jnp.float32),
            pltpu.VMEM((16,), jnp.float32),
            pltpu.VMEM((16,), jnp.int32),
            pltpu.VMEM((16,), jnp.int32),
            pltpu.VMEM((16,), jnp.float32),
            pltpu.VMEM((16,), jnp.int32),
            pltpu.VMEM((16,), jnp.int32),
            pltpu.VMEM((144,), jnp.int32),
            pltpu.VMEM((16,), jnp.int32),
            pltpu.VMEM((528,), jnp.int32),
            pltpu.VMEM((16,), jnp.float32),
            pltpu.VMEM((16,), jnp.int32),
        ],
    )(_sc_body)
    return fn(data_bt, data_length)


def kernel(data, data_length):
    data_bt = jnp.transpose(data, (1, 0, 2))
    probs8, lens8, labels = _sc_decode(data_bt, data_length.astype(jnp.int32))
    return probs8[:, :1], lens8[:, :1], labels.reshape(B, 1, T)
